# baseline (speedup 1.0000x reference)
import jax
import jax.numpy as jnp
from jax import lax
from jax.experimental import pallas as pl
from jax.experimental.pallas import tpu as pltpu
from jax.experimental.pallas import tpu_sc as plsc

_N_TOKENS = 32768
_TAB_ROWS = 4096
_WIDTHS = (16, 24, 24)
_COS_OFF = (0, 16, 40)
_SIN_OFF = (64, 80, 104)
_FUSED_OFF = (0, 32, 80)
_OUT_D = 128

_NUM_WORKERS = 32
_TOK_PER_W = _N_TOKENS // _NUM_WORKERS
_CHUNK = 128
_CHUNKS_PER_W = _TOK_PER_W // _CHUNK
_STAGE_ROWS = _TAB_ROWS // 16


def _body(ids_hbm, tab_hbm, out_hbm, idx_v, bufs0, bufs1, stabs, gsem, ssem):
    buf_sets = (bufs0, bufs1)

    c = lax.axis_index("c")
    s = lax.axis_index("s")
    wid = s * 2 + c
    base = wid * _TOK_PER_W

    rs = s * _STAGE_ROWS
    for a in range(3):
        pltpu.async_copy(
            tab_hbm.at[pl.ds(rs, _STAGE_ROWS),
                       pl.ds(_FUSED_OFF[a], 2 * _WIDTHS[a])],
            stabs[a].at[pl.ds(rs, _STAGE_ROWS), :], gsem)
    pltpu.sync_copy(ids_hbm.at[:, pl.ds(wid * _CHUNKS_PER_W, _CHUNKS_PER_W), :],
                    idx_v)
    for a in range(3):
        pltpu.make_async_copy(
            tab_hbm.at[pl.ds(rs, _STAGE_ROWS),
                       pl.ds(_FUSED_OFF[a], 2 * _WIDTHS[a])],
            stabs[a].at[pl.ds(rs, _STAGE_ROWS), :], gsem).wait()
    plsc.subcore_barrier()

    def gathers(j, bset):
        for a in range(3):
            pltpu.async_copy(stabs[a].at[idx_v.at[a, j]], bset[a], gsem)

    def wait_gathers(j, bset):
        for a in range(3):
            pltpu.make_async_copy(stabs[a].at[idx_v.at[a, j]],
                                  bset[a], gsem).wait()

    def _scatter_list(j, bset):
        rows = out_hbm.at[pl.ds(base + j * _CHUNK, _CHUNK), :]
        out = []
        for a in range(3):
            w = _WIDTHS[a]
            out.append((bset[a].at[:, pl.ds(0, w)],
                        rows.at[:, pl.ds(_COS_OFF[a], w)]))
            out.append((bset[a].at[:, pl.ds(w, w)],
                        rows.at[:, pl.ds(_SIN_OFF[a], w)]))
        return out

    def scatters(j, bset):
        for src, dst in _scatter_list(j, bset):
            pltpu.async_copy(src, dst, ssem)

    def wait_scatters(j, bset):
        for src, dst in _scatter_list(j, bset):
            pltpu.make_async_copy(src, dst, ssem).wait()

    gathers(0, buf_sets[0])
    for j in range(_CHUNKS_PER_W):
        bset = buf_sets[j % 2]
        if j + 1 < _CHUNKS_PER_W:
            if j >= 1:
                wait_scatters(j - 1, buf_sets[(j + 1) % 2])
            gathers(j + 1, buf_sets[(j + 1) % 2])
        wait_gathers(j, bset)
        scatters(j, bset)
    wait_scatters(_CHUNKS_PER_W - 2, buf_sets[(_CHUNKS_PER_W - 2) % 2])
    wait_scatters(_CHUNKS_PER_W - 1, buf_sets[(_CHUNKS_PER_W - 1) % 2])


@jax.jit
def kernel(ids, cos_0, sin_0, cos_1, sin_1, cos_2, sin_2):
    ids_r = jnp.transpose(ids.astype(jnp.int32)).reshape(
        3, _N_TOKENS // _CHUNK, _CHUNK)
    tab = jnp.concatenate([cos_0, sin_0, cos_1, sin_1, cos_2, sin_2], axis=1)

    mesh = plsc.VectorSubcoreMesh(core_axis_name="c", subcore_axis_name="s")
    run = pl.kernel(
        _body,
        out_type=jax.ShapeDtypeStruct((_N_TOKENS, _OUT_D), jnp.float32),
        mesh=mesh,
        scratch_types=[
            pltpu.VMEM((3, _CHUNKS_PER_W, _CHUNK), jnp.int32),
            tuple(pltpu.VMEM((_CHUNK, 2 * w), jnp.float32) for w in _WIDTHS),
            tuple(pltpu.VMEM((_CHUNK, 2 * w), jnp.float32) for w in _WIDTHS),
            tuple(pltpu.VMEM_SHARED((_TAB_ROWS, 2 * w), jnp.float32)
                  for w in _WIDTHS),
            pltpu.SemaphoreType.DMA,
            pltpu.SemaphoreType.DMA,
        ],
        compiler_params=pltpu.CompilerParams(use_tc_tiling_on_sc=False),
    )
    return run(ids_r, tab)


# speedup vs baseline: 1.6339x; 1.6339x over previous
import jax
import jax.numpy as jnp
from jax import lax
from jax.experimental import pallas as pl
from jax.experimental.pallas import tpu as pltpu
from jax.experimental.pallas import tpu_sc as plsc

_N_TOKENS = 32768
_TAB_ROWS = 4096
_WIDTHS = (16, 24, 24)
_COS_OFF = (0, 16, 40)
_SIN_OFF = (64, 80, 104)
_OUT_D = 128

_NUM_WORKERS = 32
_TOK_PER_W = _N_TOKENS // _NUM_WORKERS
_CHUNK = 128
_CHUNKS_PER_W = _TOK_PER_W // _CHUNK
_STAGE_ROWS = _TAB_ROWS // 16


def _body(ids_hbm, cos_0, sin_0, cos_1, sin_1, cos_2, sin_2, out_hbm,
          idx_v, bufs0, bufs1, stabs, gsem, ssem):
    tabs_hbm = (cos_0, sin_0, cos_1, sin_1, cos_2, sin_2)
    offs = (_COS_OFF[0], _SIN_OFF[0], _COS_OFF[1], _SIN_OFF[1],
            _COS_OFF[2], _SIN_OFF[2])
    axes = (0, 0, 1, 1, 2, 2)
    wids = (16, 16, 24, 24, 24, 24)
    buf_sets = (bufs0, bufs1)

    c = lax.axis_index("c")
    s = lax.axis_index("s")
    wid = s * 2 + c
    base = wid * _TOK_PER_W

    rs = s * _STAGE_ROWS
    for t in range(6):
        pltpu.async_copy(tabs_hbm[t].at[pl.ds(rs, _STAGE_ROWS), :],
                         stabs[t].at[pl.ds(rs, _STAGE_ROWS), :], gsem)
    pltpu.sync_copy(ids_hbm.at[:, pl.ds(wid * _CHUNKS_PER_W, _CHUNKS_PER_W), :],
                    idx_v)
    for t in range(6):
        pltpu.make_async_copy(tabs_hbm[t].at[pl.ds(rs, _STAGE_ROWS), :],
                              stabs[t].at[pl.ds(rs, _STAGE_ROWS), :],
                              gsem).wait()
    plsc.subcore_barrier()

    def gathers(j, bset):
        for t in range(6):
            pltpu.async_copy(stabs[t].at[idx_v.at[axes[t], j]], bset[t], gsem)

    def wait_gathers(j, bset):
        for t in range(6):
            pltpu.make_async_copy(stabs[t].at[idx_v.at[axes[t], j]],
                                  bset[t], gsem).wait()

    def out_slice(j, t):
        return out_hbm.at[pl.ds(base + j * _CHUNK, _CHUNK),
                          pl.ds(offs[t], wids[t])]

    def scatters(j, bset):
        for t in range(6):
            pltpu.async_copy(bset[t], out_slice(j, t), ssem)

    def wait_scatters(j, bset):
        for t in range(6):
            pltpu.make_async_copy(bset[t], out_slice(j, t), ssem).wait()

    gathers(0, buf_sets[0])
    for j in range(_CHUNKS_PER_W):
        bset = buf_sets[j % 2]
        if j + 1 < _CHUNKS_PER_W:
            if j >= 1:
                wait_scatters(j - 1, buf_sets[(j + 1) % 2])
            gathers(j + 1, buf_sets[(j + 1) % 2])
        wait_gathers(j, bset)
        scatters(j, bset)
    wait_scatters(_CHUNKS_PER_W - 2, buf_sets[(_CHUNKS_PER_W - 2) % 2])
    wait_scatters(_CHUNKS_PER_W - 1, buf_sets[(_CHUNKS_PER_W - 1) % 2])


@jax.jit
def kernel(ids, cos_0, sin_0, cos_1, sin_1, cos_2, sin_2):
    ids_r = jnp.transpose(ids.astype(jnp.int32)).reshape(
        3, _N_TOKENS // _CHUNK, _CHUNK)

    mesh = plsc.VectorSubcoreMesh(core_axis_name="c", subcore_axis_name="s")
    run = pl.kernel(
        _body,
        out_type=jax.ShapeDtypeStruct((_N_TOKENS, _OUT_D), jnp.float32),
        mesh=mesh,
        scratch_types=[
            pltpu.VMEM((3, _CHUNKS_PER_W, _CHUNK), jnp.int32),
            tuple(pltpu.VMEM((_CHUNK, w), jnp.float32)
                  for w in (16, 16, 24, 24, 24, 24)),
            tuple(pltpu.VMEM((_CHUNK, w), jnp.float32)
                  for w in (16, 16, 24, 24, 24, 24)),
            tuple(pltpu.VMEM_SHARED((_TAB_ROWS, w), jnp.float32)
                  for w in (16, 16, 24, 24, 24, 24)),
            pltpu.SemaphoreType.DMA,
            pltpu.SemaphoreType.DMA,
        ],
        compiler_params=pltpu.CompilerParams(use_tc_tiling_on_sc=False,
                                             needs_layout_passes=True),
    )
    return run(ids_r, cos_0, sin_0, cos_1, sin_1, cos_2, sin_2)


# speedup vs baseline: 1.6354x; 1.0010x over previous
import jax
import jax.numpy as jnp
from jax import lax
from jax.experimental import pallas as pl
from jax.experimental.pallas import tpu as pltpu
from jax.experimental.pallas import tpu_sc as plsc

_N_TOKENS = 32768
_TAB_ROWS = 4096
_WIDTHS = (16, 24, 24)
_COS_OFF = (0, 16, 40)
_SIN_OFF = (64, 80, 104)
_OUT_D = 128

_NUM_WORKERS = 32
_TOK_PER_W = _N_TOKENS // _NUM_WORKERS
_CHUNK = 128
_CHUNKS_PER_W = _TOK_PER_W // _CHUNK
_STAGE_ROWS = _TAB_ROWS // 16


def _body(ids_hbm, cos_0, sin_0, cos_1, sin_1, cos_2, sin_2, out_hbm,
          idx_v, bufs0, bufs1, bufs2, stabs, gsem, ssem):
    tabs_hbm = (cos_0, sin_0, cos_1, sin_1, cos_2, sin_2)
    offs = (_COS_OFF[0], _SIN_OFF[0], _COS_OFF[1], _SIN_OFF[1],
            _COS_OFF[2], _SIN_OFF[2])
    axes = (0, 0, 1, 1, 2, 2)
    wids = (16, 16, 24, 24, 24, 24)
    buf_sets = (bufs0, bufs1, bufs2)

    c = lax.axis_index("c")
    s = lax.axis_index("s")
    wid = s * 2 + c
    base = wid * _TOK_PER_W

    rs = s * _STAGE_ROWS
    for t in range(6):
        pltpu.async_copy(tabs_hbm[t].at[pl.ds(rs, _STAGE_ROWS), :],
                         stabs[t].at[pl.ds(rs, _STAGE_ROWS), :], gsem)
    pltpu.sync_copy(ids_hbm.at[:, pl.ds(wid * _CHUNKS_PER_W, _CHUNKS_PER_W), :],
                    idx_v)
    for t in range(6):
        pltpu.make_async_copy(tabs_hbm[t].at[pl.ds(rs, _STAGE_ROWS), :],
                              stabs[t].at[pl.ds(rs, _STAGE_ROWS), :],
                              gsem).wait()
    plsc.subcore_barrier()

    def gathers(j, bset):
        for t in range(6):
            pltpu.async_copy(stabs[t].at[idx_v.at[axes[t], j]], bset[t], gsem)

    def wait_gathers(j, bset):
        for t in range(6):
            pltpu.make_async_copy(stabs[t].at[idx_v.at[axes[t], j]],
                                  bset[t], gsem).wait()

    def out_slice(j, t):
        return out_hbm.at[pl.ds(base + j * _CHUNK, _CHUNK),
                          pl.ds(offs[t], wids[t])]

    def scatters(j, bset):
        for t in range(6):
            pltpu.async_copy(bset[t], out_slice(j, t), ssem)

    def wait_scatters(j, bset):
        for t in range(6):
            pltpu.make_async_copy(bset[t], out_slice(j, t), ssem).wait()

    gathers(0, buf_sets[0])
    gathers(1, buf_sets[1])
    for j in range(_CHUNKS_PER_W):
        bset = buf_sets[j % 3]
        if j + 2 < _CHUNKS_PER_W:
            if j >= 1:
                wait_scatters(j - 1, buf_sets[(j + 2) % 3])
            gathers(j + 2, buf_sets[(j + 2) % 3])
        wait_gathers(j, bset)
        scatters(j, bset)
    for j in (_CHUNKS_PER_W - 3, _CHUNKS_PER_W - 2, _CHUNKS_PER_W - 1):
        wait_scatters(j, buf_sets[j % 3])


@jax.jit
def kernel(ids, cos_0, sin_0, cos_1, sin_1, cos_2, sin_2):
    ids_r = jnp.transpose(ids.astype(jnp.int32)).reshape(
        3, _N_TOKENS // _CHUNK, _CHUNK)

    mesh = plsc.VectorSubcoreMesh(core_axis_name="c", subcore_axis_name="s")
    run = pl.kernel(
        _body,
        out_type=jax.ShapeDtypeStruct((_N_TOKENS, _OUT_D), jnp.float32),
        mesh=mesh,
        scratch_types=[
            pltpu.VMEM((3, _CHUNKS_PER_W, _CHUNK), jnp.int32),
            tuple(pltpu.VMEM((_CHUNK, w), jnp.float32)
                  for w in (16, 16, 24, 24, 24, 24)),
            tuple(pltpu.VMEM((_CHUNK, w), jnp.float32)
                  for w in (16, 16, 24, 24, 24, 24)),
            tuple(pltpu.VMEM((_CHUNK, w), jnp.float32)
                  for w in (16, 16, 24, 24, 24, 24)),
            tuple(pltpu.VMEM_SHARED((_TAB_ROWS, w), jnp.float32)
                  for w in (16, 16, 24, 24, 24, 24)),
            pltpu.SemaphoreType.DMA,
            pltpu.SemaphoreType.DMA,
        ],
        compiler_params=pltpu.CompilerParams(use_tc_tiling_on_sc=False),
    )
    return run(ids_r, cos_0, sin_0, cos_1, sin_1, cos_2, sin_2)
